# baseline (speedup 1.0000x reference)
import functools

import jax
import jax.numpy as jnp
from jax import lax
from jax.experimental import pallas as pl
from jax.experimental.pallas import tpu as pltpu
from jax.experimental.pallas import tpu_sc as plsc

_CH = 256
_NCHUNK = 13
_WAVE = 2048
_EW = 128


@functools.cache
def _make_kernel(B, V, D):
    info = plsc.get_sparse_core_info()
    L = info.num_lanes
    NC = info.num_cores
    NW = NC * info.num_subcores
    n_waves = B // _WAVE
    v_pad_lines = ((V + 127) // 128) * 128
    c0_max = v_pad_lines - _CH
    out_rows = B + 64
    mesh = plsc.VectorSubcoreMesh(core_axis_name="c", subcore_axis_name="s")

    @functools.partial(
        pl.kernel,
        mesh=mesh,
        compiler_params=pltpu.CompilerParams(needs_layout_passes=False),
        out_type=jax.ShapeDtypeStruct((out_rows, 2 * D), jnp.float32),
        scratch_types=[
            pltpu.VMEM((B,), jnp.int32),
            pltpu.VMEM((B,), jnp.int32),
            pltpu.VMEM((B,), jnp.int32),
            pltpu.VMEM((_WAVE,), jnp.int32),
            pltpu.VMEM((_WAVE,), jnp.int32),
            pltpu.VMEM((D, _CH), jnp.float32),
            pltpu.VMEM((_EW, 2 * D), jnp.float32),
            pltpu.VMEM((_EW,), jnp.int32),
            pltpu.SMEM((n_waves + 2,), jnp.int32),
            pltpu.SemaphoreType.DMA,
        ],
    )
    def k(idx_hbm, tt_hbm, out_hbm, idx_v, l1c, l1p, l2c, l2p, chunk, stage,
          posw, cnt, sem):
        wid = lax.axis_index("s") * NC + lax.axis_index("c")
        lanes = lax.iota(jnp.int32, L)
        lo = wid * (_NCHUNK * _CH)
        hi = lo + _NCHUNK * _CH

        pltpu.sync_copy(idx_hbm, idx_v)

        def wave_scan(w, carry):
            def grp(g, off):
                p0 = w * _WAVE + g * L
                v = idx_v[pl.ds(p0, L)]
                m = (v >= lo) & (v < hi)
                cs = plsc.cumsum(m.astype(jnp.int32))
                slot = off + cs - 1
                plsc.store_scatter(l1c, [slot], v, mask=m)
                plsc.store_scatter(l1p, [slot], p0 + lanes, mask=m)
                return off + cs[L - 1]

            off_end = lax.fori_loop(0, _WAVE // L, grp, w * _WAVE)
            cnt[w] = off_end - w * _WAVE
            return carry

        lax.fori_loop(0, n_waves, wave_scan, 0)

        def do_chunk(c, carry):
            cid = wid * _NCHUNK + c
            c0 = cid * _CH
            c0c = jnp.minimum(c0, c0_max)
            copies = []
            for b in range(D // 8):
                copies.append(
                    pltpu.async_copy(
                        tt_hbm.at[pl.ds(b * 8, 8), pl.ds(c0c, _CH)],
                        chunk.at[pl.ds(b * 8, 8), :],
                        sem,
                    )
                )
            for cp in copies:
                cp.wait()

            def do_wave(w, carry2):
                c1 = cnt[w]

                def grp2(g, off):
                    q0 = w * _WAVE + g * L
                    vc = l1c[pl.ds(q0, L)]
                    vp = l1p[pl.ds(q0, L)]
                    valid = (g * L + lanes) < c1
                    m = valid & (vc >= c0) & (vc < c0 + _CH)
                    cs = plsc.cumsum(m.astype(jnp.int32))
                    slot = off + cs - 1
                    plsc.store_scatter(l2c, [slot], vc, mask=m)
                    plsc.store_scatter(l2p, [slot], vp, mask=m)
                    return off + cs[L - 1]

                c2 = lax.fori_loop(0, (c1 + L - 1) // L, grp2, 0)

                def ewave(e, carry3):
                    s0 = e * _EW
                    for g in range(_EW // L):
                        vc = l2c[pl.ds(s0 + g * L, L)]
                        vp = l2p[pl.ds(s0 + g * L, L)]
                        valid = (s0 + g * L + lanes) < c2
                        rc = jnp.where(valid, vc - c0c, 0)
                        pos = jnp.where(valid, vp, B)
                        posw[pl.ds(g * L, L)] = pos
                        slot = g * L + lanes
                        for d in range(D):
                            dv = jnp.full((L,), d, jnp.int32)
                            word = plsc.load_gather(chunk, [dv, rc])
                            plsc.store_scatter(stage, [slot, dv], word)
                    pltpu.sync_copy(stage, out_hbm.at[posw])
                    return carry3

                lax.fori_loop(0, (c2 + _EW - 1) // _EW, ewave, 0)
                return carry2

            lax.fori_loop(0, n_waves, do_wave, 0)
            return carry

        lax.fori_loop(0, _NCHUNK, do_chunk, 0)

    return k


def kernel(x, table):
    B = x.shape[0]
    V, D = table.shape
    out_k = _make_kernel(B, V, D)(x.astype(jnp.int32), table.T)
    return out_k[:B, :D]
